# initial kernel scaffold (unmeasured)
import jax
import jax.numpy as jnp
from jax import lax
from jax.experimental import pallas as pl
from jax.experimental.pallas import tpu as pltpu

B = 4
S_HALF = 512
K = 16 * 128
N = 4096
N_ROW_CHUNKS = 2 * B


def kernel(O, Wo):
    O_r = O.reshape(N_ROW_CHUNKS, S_HALF, K).astype(jnp.bfloat16)
    Wo_b = Wo.astype(jnp.bfloat16)

    def body(
        o_hbm,
        wo_ref,
        out_hbm,
        recv_hbm,
        o_vmem,
        send_vmem,
        recv_vmem,
        acc_vmem,
        load_sem,
        store_sem,
        send_sems,
        recv_sems,
    ):
        my_x = lax.axis_index("x")
        my_y = lax.axis_index("y")
        my_z = lax.axis_index("z")
        nbr = (my_x, 1 - my_y, my_z)

        barrier = pltpu.get_barrier_semaphore()
        pl.semaphore_signal(
            barrier, inc=1, device_id=nbr, device_id_type=pl.DeviceIdType.MESH
        )
        pl.semaphore_wait(barrier, 1)

        for b in range(B):
            j = 2 * b + (1 - my_y)
            cp = pltpu.make_async_copy(o_hbm.at[j], o_vmem, load_sem)
            cp.start()
            cp.wait()
            p = jnp.dot(
                o_vmem[...], wo_ref[...], preferred_element_type=jnp.float32
            )
            send_vmem[...] = p.astype(jnp.bfloat16)
            rdma = pltpu.make_async_remote_copy(
                src_ref=send_vmem,
                dst_ref=recv_hbm.at[b],
                send_sem=send_sems.at[b],
                recv_sem=recv_sems.at[b],
                device_id=nbr,
                device_id_type=pl.DeviceIdType.MESH,
            )
            rdma.start()
            rdma.wait()

        for b in range(B):
            j = 2 * b + my_y
            cp = pltpu.make_async_copy(o_hbm.at[j], o_vmem, load_sem)
            cp.start()
            cp.wait()
            p = jnp.dot(
                o_vmem[...], wo_ref[...], preferred_element_type=jnp.float32
            )
            cr = pltpu.make_async_copy(recv_hbm.at[b], recv_vmem, load_sem)
            cr.start()
            cr.wait()
            acc_vmem[...] = p + recv_vmem[...].astype(jnp.float32)
            st = pltpu.make_async_copy(acc_vmem, out_hbm.at[b], store_sem)
            st.start()
            st.wait()

        pl.semaphore_signal(
            barrier, inc=1, device_id=nbr, device_id_type=pl.DeviceIdType.MESH
        )
        pl.semaphore_wait(barrier, 1)

    out, _recv = pl.pallas_call(
        body,
        out_shape=[
            jax.ShapeDtypeStruct((B, S_HALF, N), jnp.float32),
            jax.ShapeDtypeStruct((B, S_HALF, N), jnp.bfloat16),
        ],
        in_specs=[
            pl.BlockSpec(memory_space=pltpu.ANY),
            pl.BlockSpec(memory_space=pltpu.VMEM),
        ],
        out_specs=[
            pl.BlockSpec(memory_space=pltpu.ANY),
            pl.BlockSpec(memory_space=pltpu.ANY),
        ],
        scratch_shapes=[
            pltpu.VMEM((S_HALF, K), jnp.bfloat16),
            pltpu.VMEM((S_HALF, N), jnp.bfloat16),
            pltpu.VMEM((S_HALF, N), jnp.bfloat16),
            pltpu.VMEM((S_HALF, N), jnp.float32),
            pltpu.SemaphoreType.DMA,
            pltpu.SemaphoreType.DMA,
            pltpu.SemaphoreType.DMA((B,)),
            pltpu.SemaphoreType.DMA((B,)),
        ],
        compiler_params=pltpu.CompilerParams(collective_id=0),
    )(O_r, Wo_b)
    return out


# baseline (device time: 377172 ns/iter reference)
import jax
import jax.numpy as jnp
from jax import lax
from jax.experimental import pallas as pl
from jax.experimental.pallas import tpu as pltpu

B = 4
S_HALF = 512
K = 16 * 128
N = 4096
N_ROW_CHUNKS = 2 * B


def kernel(O, Wo):
    O_r = O.reshape(N_ROW_CHUNKS, S_HALF, K).astype(jnp.bfloat16)
    Wo_b = Wo.astype(jnp.bfloat16)

    def body(
        o_hbm,
        wo_ref,
        out_hbm,
        recv_hbm,
        o_vmem,
        send_vmem,
        recv_vmem,
        acc_vmem,
        load_sem,
        store_sem,
        send_sems,
        recv_sems,
    ):
        my_x = lax.axis_index("x")
        my_y = lax.axis_index("y")
        my_z = lax.axis_index("z")
        nbr = (my_x, 1 - my_y, my_z)

        barrier = pltpu.get_barrier_semaphore()
        pl.semaphore_signal(
            barrier, inc=1, device_id=nbr, device_id_type=pl.DeviceIdType.MESH
        )
        pl.semaphore_wait(barrier, 1)

        for b in range(B):
            j = 2 * b + (1 - my_y)
            cp = pltpu.make_async_copy(o_hbm.at[j], o_vmem, load_sem)
            cp.start()
            cp.wait()
            p = jnp.dot(
                o_vmem[...], wo_ref[...], preferred_element_type=jnp.float32
            )
            send_vmem[...] = p.astype(jnp.bfloat16)
            rdma = pltpu.make_async_remote_copy(
                src_ref=send_vmem,
                dst_ref=recv_hbm.at[b],
                send_sem=send_sems.at[b],
                recv_sem=recv_sems.at[b],
                device_id=nbr,
                device_id_type=pl.DeviceIdType.MESH,
            )
            rdma.start()
            rdma.wait()

        for b in range(B):
            j = 2 * b + my_y
            cp = pltpu.make_async_copy(o_hbm.at[j], o_vmem, load_sem)
            cp.start()
            cp.wait()
            p = jnp.dot(
                o_vmem[...], wo_ref[...], preferred_element_type=jnp.float32
            )
            cr = pltpu.make_async_copy(recv_hbm.at[b], recv_vmem, load_sem)
            cr.start()
            cr.wait()
            acc_vmem[...] = p + recv_vmem[...].astype(jnp.float32)
            st = pltpu.make_async_copy(acc_vmem, out_hbm.at[b], store_sem)
            st.start()
            st.wait()

        pl.semaphore_signal(
            barrier, inc=1, device_id=nbr, device_id_type=pl.DeviceIdType.MESH
        )
        pl.semaphore_wait(barrier, 1)

    out, _recv = pl.pallas_call(
        body,
        out_shape=[
            jax.ShapeDtypeStruct((B, S_HALF, N), jnp.float32),
            jax.ShapeDtypeStruct((B, S_HALF, N), jnp.bfloat16),
        ],
        in_specs=[
            pl.BlockSpec(memory_space=pl.ANY),
            pl.BlockSpec(memory_space=pltpu.MemorySpace.VMEM),
        ],
        out_specs=[
            pl.BlockSpec(memory_space=pl.ANY),
            pl.BlockSpec(memory_space=pl.ANY),
        ],
        scratch_shapes=[
            pltpu.VMEM((S_HALF, K), jnp.bfloat16),
            pltpu.VMEM((S_HALF, N), jnp.bfloat16),
            pltpu.VMEM((S_HALF, N), jnp.bfloat16),
            pltpu.VMEM((S_HALF, N), jnp.float32),
            pltpu.SemaphoreType.DMA,
            pltpu.SemaphoreType.DMA,
            pltpu.SemaphoreType.DMA((B,)),
            pltpu.SemaphoreType.DMA((B,)),
        ],
        compiler_params=pltpu.CompilerParams(
            collective_id=0, vmem_limit_bytes=60 * 1024 * 1024
        ),
    )(O_r, Wo_b)
    return out


# device time: 286260 ns/iter; 1.3176x vs baseline; 1.3176x over previous
import jax
import jax.numpy as jnp
from jax import lax
from jax.experimental import pallas as pl
from jax.experimental.pallas import tpu as pltpu

B = 4
S_HALF = 512
K = 16 * 128
N = 4096
N_ROW_CHUNKS = 2 * B


def kernel(O, Wo):
    O_r = O.reshape(N_ROW_CHUNKS, S_HALF, K).astype(jnp.bfloat16)
    Wo_b = Wo.astype(jnp.bfloat16)

    def body(
        o_hbm,
        wo_ref,
        out_hbm,
        recv_hbm,
        o_vmem,
        send_vmem,
        recv_vmem,
        acc_vmem,
        load_sem,
        store_sem,
        send_sems,
        recv_sems,
    ):
        my_x = lax.axis_index("x")
        my_y = lax.axis_index("y")
        my_z = lax.axis_index("z")
        nbr = (my_x, 1 - my_y, my_z)

        barrier = pltpu.get_barrier_semaphore()
        pl.semaphore_signal(
            barrier, inc=1, device_id=nbr, device_id_type=pl.DeviceIdType.MESH
        )
        pl.semaphore_wait(barrier, 1)

        rdmas = []
        for b in range(B):
            j = 2 * b + (1 - my_y)
            cp = pltpu.make_async_copy(o_hbm.at[j], o_vmem, load_sem)
            cp.start()
            cp.wait()
            p = jnp.dot(
                o_vmem[...], wo_ref[...], preferred_element_type=jnp.float32
            )
            send_vmem[b, :, :] = p.astype(jnp.bfloat16)
            rdma = pltpu.make_async_remote_copy(
                src_ref=send_vmem.at[b],
                dst_ref=recv_hbm.at[b],
                send_sem=send_sems.at[b],
                recv_sem=recv_sems.at[b],
                device_id=nbr,
                device_id_type=pl.DeviceIdType.MESH,
            )
            rdma.start()
            rdmas.append(rdma)

        for b in range(B):
            j = 2 * b + my_y
            cp = pltpu.make_async_copy(o_hbm.at[j], o_vmem, load_sem)
            cp.start()
            cp.wait()
            p = jnp.dot(
                o_vmem[...], wo_ref[...], preferred_element_type=jnp.float32
            )
            rdmas[b].wait_recv()
            cr = pltpu.make_async_copy(recv_hbm.at[b], recv_vmem, load_sem)
            cr.start()
            cr.wait()
            acc_vmem[...] = p + recv_vmem[...].astype(jnp.float32)
            st = pltpu.make_async_copy(acc_vmem, out_hbm.at[b], store_sem)
            st.start()
            st.wait()

        for b in range(B):
            rdmas[b].wait_send()

        pl.semaphore_signal(
            barrier, inc=1, device_id=nbr, device_id_type=pl.DeviceIdType.MESH
        )
        pl.semaphore_wait(barrier, 1)

    out, _recv = pl.pallas_call(
        body,
        out_shape=[
            jax.ShapeDtypeStruct((B, S_HALF, N), jnp.float32),
            jax.ShapeDtypeStruct((B, S_HALF, N), jnp.bfloat16),
        ],
        in_specs=[
            pl.BlockSpec(memory_space=pl.ANY),
            pl.BlockSpec(memory_space=pltpu.MemorySpace.VMEM),
        ],
        out_specs=[
            pl.BlockSpec(memory_space=pl.ANY),
            pl.BlockSpec(memory_space=pl.ANY),
        ],
        scratch_shapes=[
            pltpu.VMEM((S_HALF, K), jnp.bfloat16),
            pltpu.VMEM((B, S_HALF, N), jnp.bfloat16),
            pltpu.VMEM((S_HALF, N), jnp.bfloat16),
            pltpu.VMEM((S_HALF, N), jnp.float32),
            pltpu.SemaphoreType.DMA,
            pltpu.SemaphoreType.DMA,
            pltpu.SemaphoreType.DMA((B,)),
            pltpu.SemaphoreType.DMA((B,)),
        ],
        compiler_params=pltpu.CompilerParams(
            collective_id=0, vmem_limit_bytes=64 * 1024 * 1024
        ),
    )(O_r, Wo_b)
    return out


# device time: 247570 ns/iter; 1.5235x vs baseline; 1.1563x over previous
import jax
import jax.numpy as jnp
from jax import lax
from jax.experimental import pallas as pl
from jax.experimental.pallas import tpu as pltpu

B = 4
S = 1024
S_HALF = 512
H = 16
D = 128
K = H * D
N = 4096
ROWS = 256
NC = B * S_HALF // ROWS


def kernel(O, Wo):
    Wo_b = Wo.astype(jnp.bfloat16)

    def body(
        o_hbm,
        wo_ref,
        out_hbm,
        recv_hbm,
        o_stage,
        o_bf,
        send_vmem,
        recv_vmem,
        acc_vmem,
        load_sem,
        store_sem,
        send_sems,
        recv_sems,
    ):
        my_x = lax.axis_index("x")
        my_y = lax.axis_index("y")
        my_z = lax.axis_index("z")
        nbr = (my_x, 1 - my_y, my_z)

        barrier = pltpu.get_barrier_semaphore()
        pl.semaphore_signal(
            barrier, inc=1, device_id=nbr, device_id_type=pl.DeviceIdType.MESH
        )
        pl.semaphore_wait(barrier, 1)

        def load_chunk_static(c, half):
            b = c // 2
            s0 = half * S_HALF + (c % 2) * ROWS
            for h in range(H):
                pltpu.make_async_copy(
                    o_hbm.at[b, pl.ds(s0, ROWS), h],
                    o_stage.at[:, pl.ds(h * D, D)],
                    load_sem,
                ).start()
            for h in range(H):
                pltpu.make_async_copy(
                    o_hbm.at[b, pl.ds(s0, ROWS), h],
                    o_stage.at[:, pl.ds(h * D, D)],
                    load_sem,
                ).wait()

        def load_chunk(c, other_half):
            @pl.when(my_y == (1 if other_half else 0))
            def _():
                load_chunk_static(c, 0)

            @pl.when(my_y == (0 if other_half else 1))
            def _():
                load_chunk_static(c, 1)

        def partial_matmul():
            o_bf[...] = o_stage[...].astype(jnp.bfloat16)
            return jnp.dot(
                o_bf[...], wo_ref[...], preferred_element_type=jnp.float32
            )

        rdmas = []
        for c in range(NC):
            load_chunk(c, other_half=True)
            send_vmem[c, :, :] = partial_matmul().astype(jnp.bfloat16)
            rdma = pltpu.make_async_remote_copy(
                src_ref=send_vmem.at[c],
                dst_ref=recv_hbm.at[c],
                send_sem=send_sems.at[c],
                recv_sem=recv_sems.at[c],
                device_id=nbr,
                device_id_type=pl.DeviceIdType.MESH,
            )
            rdma.start()
            rdmas.append(rdma)

        for c in range(NC):
            load_chunk(c, other_half=False)
            p = partial_matmul()
            rdmas[c].wait_recv()
            cr = pltpu.make_async_copy(recv_hbm.at[c], recv_vmem, load_sem)
            cr.start()
            cr.wait()
            acc_vmem[...] = p + recv_vmem[...].astype(jnp.float32)
            st = pltpu.make_async_copy(
                acc_vmem, out_hbm.at[c // 2, pl.ds((c % 2) * ROWS, ROWS)],
                store_sem,
            )
            st.start()
            st.wait()

        for c in range(NC):
            rdmas[c].wait_send()

        pl.semaphore_signal(
            barrier, inc=1, device_id=nbr, device_id_type=pl.DeviceIdType.MESH
        )
        pl.semaphore_wait(barrier, 1)

    out, _recv = pl.pallas_call(
        body,
        out_shape=[
            jax.ShapeDtypeStruct((B, S_HALF, N), jnp.float32),
            jax.ShapeDtypeStruct((NC, ROWS, N), jnp.bfloat16),
        ],
        in_specs=[
            pl.BlockSpec(memory_space=pl.ANY),
            pl.BlockSpec(memory_space=pltpu.MemorySpace.VMEM),
        ],
        out_specs=[
            pl.BlockSpec(memory_space=pl.ANY),
            pl.BlockSpec(memory_space=pl.ANY),
        ],
        scratch_shapes=[
            pltpu.VMEM((ROWS, K), jnp.float32),
            pltpu.VMEM((ROWS, K), jnp.bfloat16),
            pltpu.VMEM((NC, ROWS, N), jnp.bfloat16),
            pltpu.VMEM((ROWS, N), jnp.bfloat16),
            pltpu.VMEM((ROWS, N), jnp.float32),
            pltpu.SemaphoreType.DMA,
            pltpu.SemaphoreType.DMA,
            pltpu.SemaphoreType.DMA((NC,)),
            pltpu.SemaphoreType.DMA((NC,)),
        ],
        compiler_params=pltpu.CompilerParams(
            collective_id=0, vmem_limit_bytes=64 * 1024 * 1024
        ),
    )(O, Wo_b)
    return out


# device time: 245575 ns/iter; 1.5359x vs baseline; 1.0081x over previous
import jax
import jax.numpy as jnp
from jax import lax
from jax.experimental import pallas as pl
from jax.experimental.pallas import tpu as pltpu

B = 4
S = 1024
S_HALF = 512
H = 16
D = 128
K = H * D
N = 4096
ROWS = 256
NC = B * S_HALF // ROWS


def kernel(O, Wo):
    Wo_b = Wo.astype(jnp.bfloat16)

    def body(
        o_hbm,
        wo_ref,
        out_hbm,
        o_stage,
        o_bf,
        send_vmem,
        recv_vmem,
        acc_vmem,
        load_sem,
        store_sem,
        send_sems,
        recv_sems,
    ):
        my_x = lax.axis_index("x")
        my_y = lax.axis_index("y")
        my_z = lax.axis_index("z")
        nbr = (my_x, 1 - my_y, my_z)

        barrier = pltpu.get_barrier_semaphore()
        pl.semaphore_signal(
            barrier, inc=1, device_id=nbr, device_id_type=pl.DeviceIdType.MESH
        )
        pl.semaphore_wait(barrier, 1)

        def load_chunk_static(c, half):
            b = c // 2
            s0 = half * S_HALF + (c % 2) * ROWS
            for h in range(H):
                pltpu.make_async_copy(
                    o_hbm.at[b, pl.ds(s0, ROWS), h],
                    o_stage.at[:, pl.ds(h * D, D)],
                    load_sem,
                ).start()
            for h in range(H):
                pltpu.make_async_copy(
                    o_hbm.at[b, pl.ds(s0, ROWS), h],
                    o_stage.at[:, pl.ds(h * D, D)],
                    load_sem,
                ).wait()

        def load_chunk(c, other_half):
            @pl.when(my_y == (1 if other_half else 0))
            def _():
                load_chunk_static(c, 0)

            @pl.when(my_y == (0 if other_half else 1))
            def _():
                load_chunk_static(c, 1)

        def partial_matmul():
            o_bf[...] = o_stage[...].astype(jnp.bfloat16)
            return jnp.dot(
                o_bf[...], wo_ref[...], preferred_element_type=jnp.float32
            )

        rdmas = []
        for c in range(NC):
            load_chunk(c, other_half=True)
            if c >= 4:
                rdmas[c - 4].wait_send()
            send_vmem[c % 4, :, :] = partial_matmul().astype(jnp.bfloat16)
            rdma = pltpu.make_async_remote_copy(
                src_ref=send_vmem.at[c % 4],
                dst_ref=recv_vmem.at[c],
                send_sem=send_sems.at[c],
                recv_sem=recv_sems.at[c],
                device_id=nbr,
                device_id_type=pl.DeviceIdType.MESH,
            )
            rdma.start()
            rdmas.append(rdma)

        for c in range(NC):
            load_chunk(c, other_half=False)
            p = partial_matmul()
            rdmas[c].wait_recv()
            acc_vmem[...] = p + recv_vmem[c].astype(jnp.float32)
            st = pltpu.make_async_copy(
                acc_vmem, out_hbm.at[c // 2, pl.ds((c % 2) * ROWS, ROWS)],
                store_sem,
            )
            st.start()
            st.wait()

        for c in range(4, NC):
            rdmas[c].wait_send()

        pl.semaphore_signal(
            barrier, inc=1, device_id=nbr, device_id_type=pl.DeviceIdType.MESH
        )
        pl.semaphore_wait(barrier, 1)

    out = pl.pallas_call(
        body,
        out_shape=jax.ShapeDtypeStruct((B, S_HALF, N), jnp.float32),
        in_specs=[
            pl.BlockSpec(memory_space=pl.ANY),
            pl.BlockSpec(memory_space=pltpu.MemorySpace.VMEM),
        ],
        out_specs=pl.BlockSpec(memory_space=pl.ANY),
        scratch_shapes=[
            pltpu.VMEM((ROWS, K), jnp.float32),
            pltpu.VMEM((ROWS, K), jnp.bfloat16),
            pltpu.VMEM((4, ROWS, N), jnp.bfloat16),
            pltpu.VMEM((NC, ROWS, N), jnp.bfloat16),
            pltpu.VMEM((ROWS, N), jnp.float32),
            pltpu.SemaphoreType.DMA,
            pltpu.SemaphoreType.DMA,
            pltpu.SemaphoreType.DMA((NC,)),
            pltpu.SemaphoreType.DMA((NC,)),
        ],
        compiler_params=pltpu.CompilerParams(
            collective_id=0, vmem_limit_bytes=64 * 1024 * 1024
        ),
    )(O, Wo_b)
    return out


# device time: 245514 ns/iter; 1.5363x vs baseline; 1.0002x over previous
import jax
import jax.numpy as jnp
from jax import lax
from jax.experimental import pallas as pl
from jax.experimental.pallas import tpu as pltpu

B = 4
S = 1024
S_HALF = 512
H = 16
D = 128
K = H * D
N = 4096
ROWS = 256
NC = B * S_HALF // ROWS


def kernel(O, Wo):
    Wo_b = Wo.astype(jnp.bfloat16)
    O_r = O.reshape(2 * B, S_HALF, H, D)

    def body(
        o_hbm,
        wo_ref,
        out_hbm,
        o_stage,
        o_bf,
        send_vmem,
        recv_vmem,
        acc_vmem,
        load_sem,
        store_sem,
        send_sems,
        recv_sems,
    ):
        my_x = lax.axis_index("x")
        my_y = lax.axis_index("y")
        my_z = lax.axis_index("z")
        nbr = (my_x, 1 - my_y, my_z)

        barrier = pltpu.get_barrier_semaphore()
        pl.semaphore_signal(
            barrier, inc=1, device_id=nbr, device_id_type=pl.DeviceIdType.MESH
        )
        pl.semaphore_wait(barrier, 1)

        def load_chunk(c, other_half):
            half = (1 - my_y) if other_half else my_y
            j = 2 * (c // 2) + half
            s0 = (c % 2) * ROWS
            for h in range(H):
                pltpu.make_async_copy(
                    o_hbm.at[j, pl.ds(s0, ROWS), h],
                    o_stage.at[:, pl.ds(h * D, D)],
                    load_sem,
                ).start()
            for h in range(H):
                pltpu.make_async_copy(
                    o_hbm.at[j, pl.ds(s0, ROWS), h],
                    o_stage.at[:, pl.ds(h * D, D)],
                    load_sem,
                ).wait()

        def partial_matmul():
            o_bf[...] = o_stage[...].astype(jnp.bfloat16)
            return jnp.dot(
                o_bf[...], wo_ref[...], preferred_element_type=jnp.float32
            )

        rdmas = []
        for c in range(NC):
            load_chunk(c, other_half=True)
            if c >= 4:
                rdmas[c - 4].wait_send()
            send_vmem[c % 4, :, :] = partial_matmul().astype(jnp.bfloat16)
            rdma = pltpu.make_async_remote_copy(
                src_ref=send_vmem.at[c % 4],
                dst_ref=recv_vmem.at[c],
                send_sem=send_sems.at[c],
                recv_sem=recv_sems.at[c],
                device_id=nbr,
                device_id_type=pl.DeviceIdType.MESH,
            )
            rdma.start()
            rdmas.append(rdma)

        for c in range(NC):
            load_chunk(c, other_half=False)
            p = partial_matmul()
            rdmas[c].wait_recv()
            acc_vmem[...] = p + recv_vmem[c].astype(jnp.float32)
            st = pltpu.make_async_copy(
                acc_vmem, out_hbm.at[c // 2, pl.ds((c % 2) * ROWS, ROWS)],
                store_sem,
            )
            st.start()
            st.wait()

        for c in range(4, NC):
            rdmas[c].wait_send()

        pl.semaphore_signal(
            barrier, inc=1, device_id=nbr, device_id_type=pl.DeviceIdType.MESH
        )
        pl.semaphore_wait(barrier, 1)

    out = pl.pallas_call(
        body,
        out_shape=jax.ShapeDtypeStruct((B, S_HALF, N), jnp.float32),
        in_specs=[
            pl.BlockSpec(memory_space=pl.ANY),
            pl.BlockSpec(memory_space=pltpu.MemorySpace.VMEM),
        ],
        out_specs=pl.BlockSpec(memory_space=pl.ANY),
        scratch_shapes=[
            pltpu.VMEM((ROWS, K), jnp.float32),
            pltpu.VMEM((ROWS, K), jnp.bfloat16),
            pltpu.VMEM((4, ROWS, N), jnp.bfloat16),
            pltpu.VMEM((NC, ROWS, N), jnp.bfloat16),
            pltpu.VMEM((ROWS, N), jnp.float32),
            pltpu.SemaphoreType.DMA,
            pltpu.SemaphoreType.DMA,
            pltpu.SemaphoreType.DMA((NC,)),
            pltpu.SemaphoreType.DMA((NC,)),
        ],
        compiler_params=pltpu.CompilerParams(
            collective_id=0, vmem_limit_bytes=64 * 1024 * 1024
        ),
    )(O_r, Wo_b)
    return out
